# baseline (speedup 1.0000x reference)
import functools

import jax
import jax.numpy as jnp
from jax import lax
from jax.experimental import pallas as pl
from jax.experimental.pallas import tpu as pltpu
from jax.experimental.pallas import tpu_sc as plsc

NC = 2
NS = 16
NW = NC * NS

B_EDGE = 6400
W_WIN = 512
BM = 400


NBUF = 5


@functools.lru_cache(maxsize=None)
def _sc_gather_fn(n_rows, d, dtype_name, n_idx, ch):
    del n_rows
    dtype = jnp.dtype(dtype_name)
    per_w = n_idx // NW
    rounds = per_w // ch
    mesh = plsc.VectorSubcoreMesh(core_axis_name="c", subcore_axis_name="s")

    @functools.partial(
        pl.kernel,
        mesh=mesh,
        out_type=jax.ShapeDtypeStruct((n_idx, d), dtype),
        scratch_types=[
            pltpu.VMEM((per_w,), jnp.int32),
            pltpu.VMEM((NBUF, ch, d), dtype),
            pltpu.SemaphoreType.DMA((NBUF,)),
            pltpu.SemaphoreType.DMA((NBUF,)),
        ],
    )
    def gather(table_hbm, idx_hbm, out_hbm, idx_v, rows_v, gsem, wsem):
        wid = lax.axis_index("s") * NC + lax.axis_index("c")
        base = wid * per_w
        pltpu.sync_copy(idx_hbm.at[pl.ds(base, per_w)], idx_v)

        def body(r0, carry):
            for b in range(NBUF):
                pltpu.async_copy(
                    table_hbm.at[idx_v.at[pl.ds((r0 + b) * ch, ch)]],
                    rows_v.at[b], gsem.at[b])
            for b in range(NBUF):
                pltpu.make_async_copy(
                    table_hbm.at[idx_v.at[pl.ds((r0 + b) * ch, ch)]],
                    rows_v.at[b], gsem.at[b]).wait()
                pltpu.async_copy(
                    rows_v.at[b],
                    out_hbm.at[pl.ds(base + (r0 + b) * ch, ch)], wsem.at[b])
            for b in range(NBUF):
                pltpu.make_async_copy(
                    rows_v.at[b],
                    out_hbm.at[pl.ds(base + (r0 + b) * ch, ch)],
                    wsem.at[b]).wait()
            return carry

        lax.fori_loop(0, rounds // NBUF, lambda i, c: body(i * NBUF, c), 0)

    return gather


def _sc_gather(table, idx, ch=40):
    r, d = table.shape
    return _sc_gather_fn(r, d, table.dtype.name, idx.shape[0], ch)(table, idx)


def _pack_bf16_words(h):
    hh = h.shape[1] // 2
    hb = h.astype(jnp.bfloat16)
    lo = lax.bitcast_convert_type(hb[:, :hh], jnp.uint16).astype(jnp.int32)
    hi = lax.bitcast_convert_type(hb[:, hh:], jnp.uint16).astype(jnp.int32)
    return lo | (hi << 16)


def _unpack_bf16_words(w):
    lo = lax.bitcast_convert_type(w << 16, jnp.float32)
    hi = lax.bitcast_convert_type(w & jnp.int32(-65536), jnp.float32)
    return lo, hi


@functools.lru_cache(maxsize=None)
def _agg_fn(n_edges, h_dim, e_dim, n_pad):
    nb = n_edges // B_EDGE

    def body(dst_s, dst_v, hg_ref, ep_ref, we_ref, be_ref, o_ref):
        blk = pl.program_id(0)

        @pl.when(blk == 0)
        def _():
            o_ref[...] = jnp.zeros_like(o_ref)

        dstv = dst_v[0, 0, :]
        hh = h_dim // 2
        glo, ghi = _unpack_bf16_words(hg_ref[...])
        ee = (jnp.dot(ep_ref[...], we_ref[...],
                      preferred_element_type=jnp.float32) + be_ref[...])
        m_lo = jnp.maximum(glo + ee[:, :hh], 0.0).astype(jnp.bfloat16)
        m_hi = jnp.maximum(ghi + ee[:, hh:], 0.0).astype(jnp.bfloat16)
        iota_b = lax.broadcasted_iota(jnp.int32, (B_EDGE,), 0)
        iota_w = lax.broadcasted_iota(jnp.int32, (W_WIN, B_EDGE), 0)

        def cond(ptr):
            return ptr < B_EDGE

        def wbody(ptr):
            base = dst_s[0, 0, ptr]
            abase = (base // 8) * 8
            lim = abase + W_WIN
            active = (iota_b >= ptr) & (dstv < lim)
            sel = ((dstv[None, :] == abase + iota_w)
                   & active[None, :]).astype(jnp.bfloat16)
            part_lo = jnp.dot(sel, m_lo, preferred_element_type=jnp.float32)
            part_hi = jnp.dot(sel, m_hi, preferred_element_type=jnp.float32)
            o_ref[pl.ds(abase, W_WIN), :hh] += part_lo
            o_ref[pl.ds(abase, W_WIN), hh:] += part_hi
            return B_EDGE - jnp.sum((dstv >= lim).astype(jnp.int32))

        lax.while_loop(cond, wbody, jnp.int32(0))

    return pl.pallas_call(
        body,
        grid=(nb,),
        in_specs=[
            pl.BlockSpec((1, 1, B_EDGE), lambda b: (b, 0, 0),
                         memory_space=pltpu.SMEM),
            pl.BlockSpec((1, 1, B_EDGE), lambda b: (b, 0, 0)),
            pl.BlockSpec((B_EDGE, h_dim // 2), lambda b: (b, 0)),
            pl.BlockSpec((B_EDGE, e_dim), lambda b: (b, 0)),
            pl.BlockSpec((e_dim, h_dim), lambda b: (0, 0)),
            pl.BlockSpec((1, h_dim), lambda b: (0, 0)),
        ],
        out_specs=pl.BlockSpec((n_pad, h_dim), lambda b: (0, 0)),
        out_shape=jax.ShapeDtypeStruct((n_pad, h_dim), jnp.float32),
    )


@functools.lru_cache(maxsize=None)
def _mm_bias_fn(m, k, n):
    def body(a_ref, w_ref, b_ref, o_ref, ob_ref):
        h = (jnp.dot(a_ref[...], w_ref[...],
                     preferred_element_type=jnp.float32) + b_ref[...])
        o_ref[...] = h
        ob_ref[...] = _pack_bf16_words(h)

    return pl.pallas_call(
        body,
        grid=(m // BM,),
        in_specs=[
            pl.BlockSpec((BM, k), lambda i: (i, 0)),
            pl.BlockSpec((k, n), lambda i: (0, 0)),
            pl.BlockSpec((1, n), lambda i: (0, 0)),
        ],
        out_specs=[
            pl.BlockSpec((BM, n), lambda i: (i, 0)),
            pl.BlockSpec((BM, n // 2), lambda i: (i, 0)),
        ],
        out_shape=[
            jax.ShapeDtypeStruct((m, n), jnp.float32),
            jax.ShapeDtypeStruct((m, n // 2), jnp.int32),
        ],
    )


@functools.lru_cache(maxsize=None)
def _layer_fn(m, h_dim, has_jk):
    def body(*refs):
        if has_jk:
            (h_ref, a_ref, jk_ref, w1_ref, b1_ref, w2_ref, b2_ref,
             h2_ref, h2b_ref, jk2_ref) = refs
        else:
            (h_ref, a_ref, w1_ref, b1_ref, w2_ref, b2_ref,
             h2_ref, h2b_ref, jk2_ref) = refs
        t = jnp.maximum(
            jnp.dot(h_ref[...] + a_ref[...], w1_ref[...],
                    preferred_element_type=jnp.float32) + b1_ref[...],
            0.0,
        )
        h2 = (jnp.dot(t, w2_ref[...], preferred_element_type=jnp.float32)
              + b2_ref[...])
        h2_ref[...] = h2
        h2b_ref[...] = _pack_bf16_words(h2)
        jk2_ref[...] = (jk_ref[...] + h2) if has_jk else h2

    w_specs = [
        pl.BlockSpec((h_dim, h_dim), lambda i: (0, 0)),
        pl.BlockSpec((1, h_dim), lambda i: (0, 0)),
        pl.BlockSpec((h_dim, h_dim), lambda i: (0, 0)),
        pl.BlockSpec((1, h_dim), lambda i: (0, 0)),
    ]
    in_specs = [
        pl.BlockSpec((BM, h_dim), lambda i: (i, 0)),
        pl.BlockSpec((BM, h_dim), lambda i: (i, 0)),
    ]
    if has_jk:
        in_specs.append(pl.BlockSpec((BM, h_dim), lambda i: (i, 0)))
    in_specs += w_specs

    return pl.pallas_call(
        body,
        grid=(m // BM,),
        in_specs=in_specs,
        out_specs=[
            pl.BlockSpec((BM, h_dim), lambda i: (i, 0)),
            pl.BlockSpec((BM, h_dim // 2), lambda i: (i, 0)),
            pl.BlockSpec((BM, h_dim), lambda i: (i, 0)),
        ],
        out_shape=[
            jax.ShapeDtypeStruct((m, h_dim), jnp.float32),
            jax.ShapeDtypeStruct((m, h_dim // 2), jnp.int32),
            jax.ShapeDtypeStruct((m, h_dim), jnp.float32),
        ],
    )


@functools.lru_cache(maxsize=None)
def _readout_fn(m, h_dim, c_dim):
    def body(j1, j2, j3, w1, w2, w3, bias, o_ref):
        acc = jnp.dot(j1[...], w1[...], preferred_element_type=jnp.float32)
        acc += jnp.dot(j2[...], w2[...], preferred_element_type=jnp.float32)
        acc += jnp.dot(j3[...], w3[...], preferred_element_type=jnp.float32)
        o_ref[...] = acc + bias[...]

    jk_spec = pl.BlockSpec((BM, h_dim), lambda i: (i, 0))
    w_spec = pl.BlockSpec((h_dim, c_dim), lambda i: (0, 0))
    return pl.pallas_call(
        body,
        grid=(m // BM,),
        in_specs=[jk_spec, jk_spec, jk_spec, w_spec, w_spec, w_spec,
                  pl.BlockSpec((1, c_dim), lambda i: (0, 0))],
        out_specs=pl.BlockSpec((BM, c_dim), lambda i: (i, 0)),
        out_shape=jax.ShapeDtypeStruct((m, c_dim), jnp.float32),
    )


def kernel(x, edge_index, e, Wnode, bnode, Wedge, bedge, W1, b1, W2, b2,
           Wout, bout):
    n, nd = x.shape
    n_edges = e.shape[0]
    e_dim = e.shape[1]
    h_dim = Wnode.shape[2]
    c_dim = Wout.shape[2]
    n_members = Wnode.shape[0]
    n_layers = W1.shape[1]
    n_pad = ((n + W_WIN + 7) // 8) * 8 + 8

    src = edge_index[0]
    dst = edge_index[1]
    eidx = lax.iota(jnp.uint32, n_edges)
    key = jnp.sort(dst.astype(jnp.uint32) * jnp.uint32(1 << 18) + eidx)
    perm = (key & jnp.uint32((1 << 18) - 1)).astype(jnp.int32)
    dstp = (key >> jnp.uint32(18)).astype(jnp.int32)
    srcp = jnp.take(src, perm)
    nb = n_edges // B_EDGE
    dstp3 = dstp.reshape(nb, 1, B_EDGE)

    e_dim_p = 128
    e_padded = jnp.pad(e, ((0, 0), (0, e_dim_p - e_dim)))
    ep = _sc_gather(e_padded, perm).astype(jnp.bfloat16)

    agg_call = _agg_fn(n_edges, h_dim, e_dim_p, n_pad)
    mm_node = _mm_bias_fn(n, nd, h_dim)

    hs = [mm_node(x, Wnode[g], bnode[g].reshape(1, h_dim))
          for g in range(n_members)]
    jks = [None] * n_members
    for l in range(n_layers):
        hgs = [_sc_gather(hs[g][1], srcp) for g in range(n_members)]
        for g in range(n_members):
            we_p = jnp.pad(Wedge[g, l], ((0, e_dim_p - e_dim), (0, 0))
                           ).astype(jnp.bfloat16)
            agg = agg_call(dstp3, dstp3, hgs[g], ep, we_p,
                           bedge[g, l].reshape(1, h_dim))
            layer_call = _layer_fn(n, h_dim, jks[g] is not None)
            args = [hs[g][0], agg]
            if jks[g] is not None:
                args.append(jks[g])
            args += [W1[g, l], b1[g, l].reshape(1, h_dim),
                     W2[g, l], b2[g, l].reshape(1, h_dim)]
            h_new, hb_new, jks[g] = layer_call(*args)
            hs[g] = (h_new, hb_new)

    bsum = (bout[0] + bout[1] + bout[2]).reshape(1, c_dim)
    out = _readout_fn(n, h_dim, c_dim)(
        jks[0], jks[1], jks[2], Wout[0], Wout[1], Wout[2], bsum)
    return out


# speedup vs baseline: 1.0284x; 1.0284x over previous
import functools

import jax
import jax.numpy as jnp
from jax import lax
from jax.experimental import pallas as pl
from jax.experimental.pallas import tpu as pltpu
from jax.experimental.pallas import tpu_sc as plsc

NC = 2
NS = 16
NW = NC * NS

B_EDGE = 4000
W_WIN = 320
BM = 400


NBUF = 5


@functools.lru_cache(maxsize=None)
def _sc_gather_fn(n_rows, d, dtype_name, n_idx, ch):
    del n_rows
    dtype = jnp.dtype(dtype_name)
    per_w = n_idx // NW
    rounds = per_w // ch
    mesh = plsc.VectorSubcoreMesh(core_axis_name="c", subcore_axis_name="s")

    @functools.partial(
        pl.kernel,
        mesh=mesh,
        out_type=jax.ShapeDtypeStruct((n_idx, d), dtype),
        scratch_types=[
            pltpu.VMEM((per_w,), jnp.int32),
            pltpu.VMEM((NBUF, ch, d), dtype),
            pltpu.SemaphoreType.DMA((NBUF,)),
            pltpu.SemaphoreType.DMA((NBUF,)),
        ],
    )
    def gather(table_hbm, idx_hbm, out_hbm, idx_v, rows_v, gsem, wsem):
        wid = lax.axis_index("s") * NC + lax.axis_index("c")
        base = wid * per_w
        pltpu.sync_copy(idx_hbm.at[pl.ds(base, per_w)], idx_v)

        def body(r0, carry):
            for b in range(NBUF):
                pltpu.async_copy(
                    table_hbm.at[idx_v.at[pl.ds((r0 + b) * ch, ch)]],
                    rows_v.at[b], gsem.at[b])
            for b in range(NBUF):
                pltpu.make_async_copy(
                    table_hbm.at[idx_v.at[pl.ds((r0 + b) * ch, ch)]],
                    rows_v.at[b], gsem.at[b]).wait()
                pltpu.async_copy(
                    rows_v.at[b],
                    out_hbm.at[pl.ds(base + (r0 + b) * ch, ch)], wsem.at[b])
            for b in range(NBUF):
                pltpu.make_async_copy(
                    rows_v.at[b],
                    out_hbm.at[pl.ds(base + (r0 + b) * ch, ch)],
                    wsem.at[b]).wait()
            return carry

        lax.fori_loop(0, rounds // NBUF, lambda i, c: body(i * NBUF, c), 0)

    return gather


def _sc_gather(table, idx, ch=40):
    r, d = table.shape
    return _sc_gather_fn(r, d, table.dtype.name, idx.shape[0], ch)(table, idx)


def _pack_bf16_words(h):
    hh = h.shape[1] // 2
    hb = h.astype(jnp.bfloat16)
    lo = lax.bitcast_convert_type(hb[:, :hh], jnp.uint16).astype(jnp.int32)
    hi = lax.bitcast_convert_type(hb[:, hh:], jnp.uint16).astype(jnp.int32)
    return lo | (hi << 16)


def _unpack_bf16_words(w):
    lo = lax.bitcast_convert_type(w << 16, jnp.float32)
    hi = lax.bitcast_convert_type(w & jnp.int32(-65536), jnp.float32)
    return lo, hi


@functools.lru_cache(maxsize=None)
def _agg_fn(n_edges, h_dim, e_dim, n_pad):
    nb = n_edges // B_EDGE

    def body(dst_s, dst_v, hg_ref, ep_ref, we_ref, be_ref, o_ref):
        blk = pl.program_id(0)

        @pl.when(blk == 0)
        def _():
            o_ref[...] = jnp.zeros_like(o_ref)

        dstv = dst_v[0, 0, :]
        hh = h_dim // 2
        glo, ghi = _unpack_bf16_words(hg_ref[...])
        ee = (jnp.dot(ep_ref[...], we_ref[...],
                      preferred_element_type=jnp.float32) + be_ref[...])
        m_lo = jnp.maximum(glo + ee[:, :hh], 0.0).astype(jnp.bfloat16)
        m_hi = jnp.maximum(ghi + ee[:, hh:], 0.0).astype(jnp.bfloat16)
        iota_b = lax.broadcasted_iota(jnp.int32, (B_EDGE,), 0)
        iota_w = lax.broadcasted_iota(jnp.int32, (W_WIN, B_EDGE), 0)

        def cond(ptr):
            return ptr < B_EDGE

        def wbody(ptr):
            base = dst_s[0, 0, ptr]
            abase = (base // 8) * 8
            lim = abase + W_WIN
            active = (iota_b >= ptr) & (dstv < lim)
            sel = ((dstv[None, :] == abase + iota_w)
                   & active[None, :]).astype(jnp.bfloat16)
            part_lo = jnp.dot(sel, m_lo, preferred_element_type=jnp.float32)
            part_hi = jnp.dot(sel, m_hi, preferred_element_type=jnp.float32)
            o_ref[pl.ds(abase, W_WIN), :hh] += part_lo
            o_ref[pl.ds(abase, W_WIN), hh:] += part_hi
            return B_EDGE - jnp.sum((dstv >= lim).astype(jnp.int32))

        lax.while_loop(cond, wbody, jnp.int32(0))

    return pl.pallas_call(
        body,
        grid=(nb,),
        in_specs=[
            pl.BlockSpec((1, 1, B_EDGE), lambda b: (b, 0, 0),
                         memory_space=pltpu.SMEM),
            pl.BlockSpec((1, 1, B_EDGE), lambda b: (b, 0, 0)),
            pl.BlockSpec((B_EDGE, h_dim // 2), lambda b: (b, 0)),
            pl.BlockSpec((B_EDGE, e_dim), lambda b: (b, 0)),
            pl.BlockSpec((e_dim, h_dim), lambda b: (0, 0)),
            pl.BlockSpec((1, h_dim), lambda b: (0, 0)),
        ],
        out_specs=pl.BlockSpec((n_pad, h_dim), lambda b: (0, 0)),
        out_shape=jax.ShapeDtypeStruct((n_pad, h_dim), jnp.float32),
    )


@functools.lru_cache(maxsize=None)
def _mm_bias_fn(m, k, n):
    def body(a_ref, w_ref, b_ref, o_ref, ob_ref):
        h = (jnp.dot(a_ref[...], w_ref[...],
                     preferred_element_type=jnp.float32) + b_ref[...])
        o_ref[...] = h
        ob_ref[...] = _pack_bf16_words(h)

    return pl.pallas_call(
        body,
        grid=(m // BM,),
        in_specs=[
            pl.BlockSpec((BM, k), lambda i: (i, 0)),
            pl.BlockSpec((k, n), lambda i: (0, 0)),
            pl.BlockSpec((1, n), lambda i: (0, 0)),
        ],
        out_specs=[
            pl.BlockSpec((BM, n), lambda i: (i, 0)),
            pl.BlockSpec((BM, n // 2), lambda i: (i, 0)),
        ],
        out_shape=[
            jax.ShapeDtypeStruct((m, n), jnp.float32),
            jax.ShapeDtypeStruct((m, n // 2), jnp.int32),
        ],
    )


@functools.lru_cache(maxsize=None)
def _layer_fn(m, h_dim, has_jk):
    def body(*refs):
        if has_jk:
            (h_ref, a_ref, jk_ref, w1_ref, b1_ref, w2_ref, b2_ref,
             h2_ref, h2b_ref, jk2_ref) = refs
        else:
            (h_ref, a_ref, w1_ref, b1_ref, w2_ref, b2_ref,
             h2_ref, h2b_ref, jk2_ref) = refs
        t = jnp.maximum(
            jnp.dot(h_ref[...] + a_ref[...], w1_ref[...],
                    preferred_element_type=jnp.float32) + b1_ref[...],
            0.0,
        )
        h2 = (jnp.dot(t, w2_ref[...], preferred_element_type=jnp.float32)
              + b2_ref[...])
        h2_ref[...] = h2
        h2b_ref[...] = _pack_bf16_words(h2)
        jk2_ref[...] = (jk_ref[...] + h2) if has_jk else h2

    w_specs = [
        pl.BlockSpec((h_dim, h_dim), lambda i: (0, 0)),
        pl.BlockSpec((1, h_dim), lambda i: (0, 0)),
        pl.BlockSpec((h_dim, h_dim), lambda i: (0, 0)),
        pl.BlockSpec((1, h_dim), lambda i: (0, 0)),
    ]
    in_specs = [
        pl.BlockSpec((BM, h_dim), lambda i: (i, 0)),
        pl.BlockSpec((BM, h_dim), lambda i: (i, 0)),
    ]
    if has_jk:
        in_specs.append(pl.BlockSpec((BM, h_dim), lambda i: (i, 0)))
    in_specs += w_specs

    return pl.pallas_call(
        body,
        grid=(m // BM,),
        in_specs=in_specs,
        out_specs=[
            pl.BlockSpec((BM, h_dim), lambda i: (i, 0)),
            pl.BlockSpec((BM, h_dim // 2), lambda i: (i, 0)),
            pl.BlockSpec((BM, h_dim), lambda i: (i, 0)),
        ],
        out_shape=[
            jax.ShapeDtypeStruct((m, h_dim), jnp.float32),
            jax.ShapeDtypeStruct((m, h_dim // 2), jnp.int32),
            jax.ShapeDtypeStruct((m, h_dim), jnp.float32),
        ],
    )


@functools.lru_cache(maxsize=None)
def _readout_fn(m, h_dim, c_dim):
    def body(j1, j2, j3, w1, w2, w3, bias, o_ref):
        acc = jnp.dot(j1[...], w1[...], preferred_element_type=jnp.float32)
        acc += jnp.dot(j2[...], w2[...], preferred_element_type=jnp.float32)
        acc += jnp.dot(j3[...], w3[...], preferred_element_type=jnp.float32)
        o_ref[...] = acc + bias[...]

    jk_spec = pl.BlockSpec((BM, h_dim), lambda i: (i, 0))
    w_spec = pl.BlockSpec((h_dim, c_dim), lambda i: (0, 0))
    return pl.pallas_call(
        body,
        grid=(m // BM,),
        in_specs=[jk_spec, jk_spec, jk_spec, w_spec, w_spec, w_spec,
                  pl.BlockSpec((1, c_dim), lambda i: (0, 0))],
        out_specs=pl.BlockSpec((BM, c_dim), lambda i: (i, 0)),
        out_shape=jax.ShapeDtypeStruct((m, c_dim), jnp.float32),
    )


def kernel(x, edge_index, e, Wnode, bnode, Wedge, bedge, W1, b1, W2, b2,
           Wout, bout):
    n, nd = x.shape
    n_edges = e.shape[0]
    e_dim = e.shape[1]
    h_dim = Wnode.shape[2]
    c_dim = Wout.shape[2]
    n_members = Wnode.shape[0]
    n_layers = W1.shape[1]
    n_pad = ((n + W_WIN + 7) // 8) * 8 + 8

    src = edge_index[0]
    dst = edge_index[1]
    eidx = lax.iota(jnp.uint32, n_edges)
    key = jnp.sort(dst.astype(jnp.uint32) * jnp.uint32(1 << 18) + eidx)
    perm = (key & jnp.uint32((1 << 18) - 1)).astype(jnp.int32)
    dstp = (key >> jnp.uint32(18)).astype(jnp.int32)
    srcp = jnp.take(src, perm)
    nb = n_edges // B_EDGE
    dstp3 = dstp.reshape(nb, 1, B_EDGE)

    e_dim_p = 128
    e_padded = jnp.pad(e, ((0, 0), (0, e_dim_p - e_dim)))
    ep = _sc_gather(e_padded, perm).astype(jnp.bfloat16)

    agg_call = _agg_fn(n_edges, h_dim, e_dim_p, n_pad)
    mm_node = _mm_bias_fn(n, nd, h_dim)

    hs = [mm_node(x, Wnode[g], bnode[g].reshape(1, h_dim))
          for g in range(n_members)]
    jks = [None] * n_members
    for l in range(n_layers):
        hgs = [_sc_gather(hs[g][1], srcp) for g in range(n_members)]
        for g in range(n_members):
            we_p = jnp.pad(Wedge[g, l], ((0, e_dim_p - e_dim), (0, 0))
                           ).astype(jnp.bfloat16)
            agg = agg_call(dstp3, dstp3, hgs[g], ep, we_p,
                           bedge[g, l].reshape(1, h_dim))
            layer_call = _layer_fn(n, h_dim, jks[g] is not None)
            args = [hs[g][0], agg]
            if jks[g] is not None:
                args.append(jks[g])
            args += [W1[g, l], b1[g, l].reshape(1, h_dim),
                     W2[g, l], b2[g, l].reshape(1, h_dim)]
            h_new, hb_new, jks[g] = layer_call(*args)
            hs[g] = (h_new, hb_new)

    bsum = (bout[0] + bout[1] + bout[2]).reshape(1, c_dim)
    out = _readout_fn(n, h_dim, c_dim)(
        jks[0], jks[1], jks[2], Wout[0], Wout[1], Wout[2], bsum)
    return out


# speedup vs baseline: 1.0397x; 1.0110x over previous
import functools

import jax
import jax.numpy as jnp
from jax import lax
from jax.experimental import pallas as pl
from jax.experimental.pallas import tpu as pltpu
from jax.experimental.pallas import tpu_sc as plsc

NC = 2
NS = 16
NW = NC * NS

B_EDGE = 3200
W_WIN = 256
BM = 400


NBUF = 5


@functools.lru_cache(maxsize=None)
def _sc_gather_fn(n_rows, d, dtype_name, n_idx, ch):
    del n_rows
    dtype = jnp.dtype(dtype_name)
    per_w = n_idx // NW
    rounds = per_w // ch
    mesh = plsc.VectorSubcoreMesh(core_axis_name="c", subcore_axis_name="s")

    @functools.partial(
        pl.kernel,
        mesh=mesh,
        out_type=jax.ShapeDtypeStruct((n_idx, d), dtype),
        scratch_types=[
            pltpu.VMEM((per_w,), jnp.int32),
            pltpu.VMEM((NBUF, ch, d), dtype),
            pltpu.SemaphoreType.DMA((NBUF,)),
            pltpu.SemaphoreType.DMA((NBUF,)),
        ],
    )
    def gather(table_hbm, idx_hbm, out_hbm, idx_v, rows_v, gsem, wsem):
        wid = lax.axis_index("s") * NC + lax.axis_index("c")
        base = wid * per_w
        pltpu.sync_copy(idx_hbm.at[pl.ds(base, per_w)], idx_v)

        def body(r0, carry):
            for b in range(NBUF):
                pltpu.async_copy(
                    table_hbm.at[idx_v.at[pl.ds((r0 + b) * ch, ch)]],
                    rows_v.at[b], gsem.at[b])
            for b in range(NBUF):
                pltpu.make_async_copy(
                    table_hbm.at[idx_v.at[pl.ds((r0 + b) * ch, ch)]],
                    rows_v.at[b], gsem.at[b]).wait()
                pltpu.async_copy(
                    rows_v.at[b],
                    out_hbm.at[pl.ds(base + (r0 + b) * ch, ch)], wsem.at[b])
            for b in range(NBUF):
                pltpu.make_async_copy(
                    rows_v.at[b],
                    out_hbm.at[pl.ds(base + (r0 + b) * ch, ch)],
                    wsem.at[b]).wait()
            return carry

        lax.fori_loop(0, rounds // NBUF, lambda i, c: body(i * NBUF, c), 0)

    return gather


def _sc_gather(table, idx, ch=40):
    r, d = table.shape
    return _sc_gather_fn(r, d, table.dtype.name, idx.shape[0], ch)(table, idx)


def _pack_bf16_words(h):
    hh = h.shape[1] // 2
    hb = h.astype(jnp.bfloat16)
    lo = lax.bitcast_convert_type(hb[:, :hh], jnp.uint16).astype(jnp.int32)
    hi = lax.bitcast_convert_type(hb[:, hh:], jnp.uint16).astype(jnp.int32)
    return lo | (hi << 16)


def _unpack_bf16_words(w):
    lo = lax.bitcast_convert_type(w << 16, jnp.float32)
    hi = lax.bitcast_convert_type(w & jnp.int32(-65536), jnp.float32)
    return lo, hi


@functools.lru_cache(maxsize=None)
def _agg_fn(n_edges, h_dim, e_dim, e_dim_true, n_pad):
    nb = n_edges // B_EDGE

    def body(dst_s, dst_v, hg_ref, ep_ref, we_ref, be_ref, o_ref):
        blk = pl.program_id(0)

        @pl.when(blk == 0)
        def _():
            o_ref[...] = jnp.zeros_like(o_ref)

        dstv = dst_v[0, 0, :]
        hh = h_dim // 2
        glo, ghi = _unpack_bf16_words(hg_ref[...])
        ee = (jnp.dot(ep_ref[..., :e_dim_true], we_ref[...],
                      preferred_element_type=jnp.float32) + be_ref[...])
        m_lo = jnp.maximum(glo + ee[:, :hh], 0.0).astype(jnp.bfloat16)
        m_hi = jnp.maximum(ghi + ee[:, hh:], 0.0).astype(jnp.bfloat16)
        iota_b = lax.broadcasted_iota(jnp.int32, (B_EDGE,), 0)
        iota_w = lax.broadcasted_iota(jnp.int32, (W_WIN, B_EDGE), 0)

        def cond(ptr):
            return ptr < B_EDGE

        def wbody(ptr):
            base = dst_s[0, 0, ptr]
            abase = (base // 8) * 8
            lim = abase + W_WIN
            active = (iota_b >= ptr) & (dstv < lim)
            sel = ((dstv[None, :] == abase + iota_w)
                   & active[None, :]).astype(jnp.bfloat16)
            part_lo = jnp.dot(sel, m_lo, preferred_element_type=jnp.float32)
            part_hi = jnp.dot(sel, m_hi, preferred_element_type=jnp.float32)
            o_ref[pl.ds(abase, W_WIN), :hh] += part_lo
            o_ref[pl.ds(abase, W_WIN), hh:] += part_hi
            return B_EDGE - jnp.sum((dstv >= lim).astype(jnp.int32))

        lax.while_loop(cond, wbody, jnp.int32(0))

    return pl.pallas_call(
        body,
        grid=(nb,),
        in_specs=[
            pl.BlockSpec((1, 1, B_EDGE), lambda b: (b, 0, 0),
                         memory_space=pltpu.SMEM),
            pl.BlockSpec((1, 1, B_EDGE), lambda b: (b, 0, 0)),
            pl.BlockSpec((B_EDGE, h_dim // 2), lambda b: (b, 0)),
            pl.BlockSpec((B_EDGE, e_dim), lambda b: (b, 0)),
            pl.BlockSpec((e_dim_true, h_dim), lambda b: (0, 0)),
            pl.BlockSpec((1, h_dim), lambda b: (0, 0)),
        ],
        out_specs=pl.BlockSpec((n_pad, h_dim), lambda b: (0, 0)),
        out_shape=jax.ShapeDtypeStruct((n_pad, h_dim), jnp.float32),
    )


@functools.lru_cache(maxsize=None)
def _mm_bias_fn(m, k, n):
    def body(a_ref, w_ref, b_ref, o_ref, ob_ref):
        h = (jnp.dot(a_ref[...], w_ref[...],
                     preferred_element_type=jnp.float32) + b_ref[...])
        o_ref[...] = h
        ob_ref[...] = _pack_bf16_words(h)

    return pl.pallas_call(
        body,
        grid=(m // BM,),
        in_specs=[
            pl.BlockSpec((BM, k), lambda i: (i, 0)),
            pl.BlockSpec((k, n), lambda i: (0, 0)),
            pl.BlockSpec((1, n), lambda i: (0, 0)),
        ],
        out_specs=[
            pl.BlockSpec((BM, n), lambda i: (i, 0)),
            pl.BlockSpec((BM, n // 2), lambda i: (i, 0)),
        ],
        out_shape=[
            jax.ShapeDtypeStruct((m, n), jnp.float32),
            jax.ShapeDtypeStruct((m, n // 2), jnp.int32),
        ],
    )


@functools.lru_cache(maxsize=None)
def _layer_fn(m, h_dim, has_jk):
    def body(*refs):
        if has_jk:
            (h_ref, a_ref, jk_ref, w1_ref, b1_ref, w2_ref, b2_ref,
             h2_ref, h2b_ref, jk2_ref) = refs
        else:
            (h_ref, a_ref, w1_ref, b1_ref, w2_ref, b2_ref,
             h2_ref, h2b_ref, jk2_ref) = refs
        t = jnp.maximum(
            jnp.dot(h_ref[...] + a_ref[...], w1_ref[...],
                    preferred_element_type=jnp.float32) + b1_ref[...],
            0.0,
        )
        h2 = (jnp.dot(t, w2_ref[...], preferred_element_type=jnp.float32)
              + b2_ref[...])
        h2_ref[...] = h2
        h2b_ref[...] = _pack_bf16_words(h2)
        jk2_ref[...] = (jk_ref[...] + h2) if has_jk else h2

    w_specs = [
        pl.BlockSpec((h_dim, h_dim), lambda i: (0, 0)),
        pl.BlockSpec((1, h_dim), lambda i: (0, 0)),
        pl.BlockSpec((h_dim, h_dim), lambda i: (0, 0)),
        pl.BlockSpec((1, h_dim), lambda i: (0, 0)),
    ]
    in_specs = [
        pl.BlockSpec((BM, h_dim), lambda i: (i, 0)),
        pl.BlockSpec((BM, h_dim), lambda i: (i, 0)),
    ]
    if has_jk:
        in_specs.append(pl.BlockSpec((BM, h_dim), lambda i: (i, 0)))
    in_specs += w_specs

    return pl.pallas_call(
        body,
        grid=(m // BM,),
        in_specs=in_specs,
        out_specs=[
            pl.BlockSpec((BM, h_dim), lambda i: (i, 0)),
            pl.BlockSpec((BM, h_dim // 2), lambda i: (i, 0)),
            pl.BlockSpec((BM, h_dim), lambda i: (i, 0)),
        ],
        out_shape=[
            jax.ShapeDtypeStruct((m, h_dim), jnp.float32),
            jax.ShapeDtypeStruct((m, h_dim // 2), jnp.int32),
            jax.ShapeDtypeStruct((m, h_dim), jnp.float32),
        ],
    )


@functools.lru_cache(maxsize=None)
def _readout_fn(m, h_dim, c_dim):
    def body(j1, j2, j3, w1, w2, w3, bias, o_ref):
        acc = jnp.dot(j1[...], w1[...], preferred_element_type=jnp.float32)
        acc += jnp.dot(j2[...], w2[...], preferred_element_type=jnp.float32)
        acc += jnp.dot(j3[...], w3[...], preferred_element_type=jnp.float32)
        o_ref[...] = acc + bias[...]

    jk_spec = pl.BlockSpec((BM, h_dim), lambda i: (i, 0))
    w_spec = pl.BlockSpec((h_dim, c_dim), lambda i: (0, 0))
    return pl.pallas_call(
        body,
        grid=(m // BM,),
        in_specs=[jk_spec, jk_spec, jk_spec, w_spec, w_spec, w_spec,
                  pl.BlockSpec((1, c_dim), lambda i: (0, 0))],
        out_specs=pl.BlockSpec((BM, c_dim), lambda i: (i, 0)),
        out_shape=jax.ShapeDtypeStruct((m, c_dim), jnp.float32),
    )


def kernel(x, edge_index, e, Wnode, bnode, Wedge, bedge, W1, b1, W2, b2,
           Wout, bout):
    n, nd = x.shape
    n_edges = e.shape[0]
    e_dim = e.shape[1]
    h_dim = Wnode.shape[2]
    c_dim = Wout.shape[2]
    n_members = Wnode.shape[0]
    n_layers = W1.shape[1]
    n_pad = ((n + W_WIN + 7) // 8) * 8 + 8

    src = edge_index[0]
    dst = edge_index[1]
    eidx = lax.iota(jnp.uint32, n_edges)
    key = jnp.sort(dst.astype(jnp.uint32) * jnp.uint32(1 << 18) + eidx)
    perm = (key & jnp.uint32((1 << 18) - 1)).astype(jnp.int32)
    dstp = (key >> jnp.uint32(18)).astype(jnp.int32)
    srcp = jnp.take(src, perm)
    nb = n_edges // B_EDGE
    dstp3 = dstp.reshape(nb, 1, B_EDGE)

    e_dim_p = 128
    e_padded = jnp.pad(e, ((0, 0), (0, e_dim_p - e_dim)))
    ep = _sc_gather(e_padded, perm).astype(jnp.bfloat16)

    agg_call = _agg_fn(n_edges, h_dim, e_dim_p, e_dim, n_pad)
    mm_node = _mm_bias_fn(n, nd, h_dim)

    hs = [mm_node(x, Wnode[g], bnode[g].reshape(1, h_dim))
          for g in range(n_members)]
    jks = [None] * n_members
    for l in range(n_layers):
        hgs = [_sc_gather(hs[g][1], srcp) for g in range(n_members)]
        for g in range(n_members):
            agg = agg_call(dstp3, dstp3, hgs[g], ep,
                           Wedge[g, l].astype(jnp.bfloat16),
                           bedge[g, l].reshape(1, h_dim))
            layer_call = _layer_fn(n, h_dim, jks[g] is not None)
            args = [hs[g][0], agg]
            if jks[g] is not None:
                args.append(jks[g])
            args += [W1[g, l], b1[g, l].reshape(1, h_dim),
                     W2[g, l], b2[g, l].reshape(1, h_dim)]
            h_new, hb_new, jks[g] = layer_call(*args)
            hs[g] = (h_new, hb_new)

    bsum = (bout[0] + bout[1] + bout[2]).reshape(1, c_dim)
    out = _readout_fn(n, h_dim, c_dim)(
        jks[0], jks[1], jks[2], Wout[0], Wout[1], Wout[2], bsum)
    return out
